# baseline (device time: 216194 ns/iter reference)
import jax
import jax.numpy as jnp
from jax import lax
from jax.experimental import pallas as pl
from jax.experimental.pallas import tpu as pltpu

N_DEV = 16


def kernel(x, w_mat, scale_x, scale_w):
    m_per, k = x.shape
    _, n_per = w_mat.shape

    x8 = x.astype(jnp.float8_e4m3fn)
    wb = w_mat.astype(jnp.bfloat16)
    s = (scale_x * scale_w).reshape(1, 1)

    def body(x_ref, w_ref, s_ref, out_ref, gat_ref, send_sems, recv_sems):
        my = lax.axis_index("i")
        left = lax.rem(my + N_DEV - 1, N_DEV)
        right = lax.rem(my + 1, N_DEV)

        barrier_sem = pltpu.get_barrier_semaphore()
        for nbr in (left, right):
            pl.semaphore_signal(
                barrier_sem, inc=1,
                device_id=(nbr,), device_id_type=pl.DeviceIdType.MESH,
            )
        pl.semaphore_wait(barrier_sem, 2)

        gat_ref[my] = x_ref[...]

        def compute(c):
            xb = gat_ref[c].astype(jnp.bfloat16)
            acc = jnp.dot(xb, w_ref[...], preferred_element_type=jnp.float32)
            y = acc * s_ref[0, 0]
            out_ref[pl.ds(c * m_per, m_per), :] = y * (
                1.0 / (1.0 + jnp.exp(-y))
            )

        compute(my)

        for h in range(N_DEV - 1):
            c_send = lax.rem(my - h + N_DEV, N_DEV)
            rdma = pltpu.make_async_remote_copy(
                src_ref=gat_ref.at[c_send],
                dst_ref=gat_ref.at[c_send],
                send_sem=send_sems.at[h],
                recv_sem=recv_sems.at[h],
                device_id=(right,),
                device_id_type=pl.DeviceIdType.MESH,
            )
            rdma.start()
            rdma.wait()
            c_recv = lax.rem(my - h - 1 + N_DEV, N_DEV)
            compute(c_recv)

    return pl.pallas_call(
        body,
        out_shape=jax.ShapeDtypeStruct((N_DEV * m_per, n_per), jnp.float32),
        in_specs=[
            pl.BlockSpec(memory_space=pltpu.VMEM),
            pl.BlockSpec(memory_space=pltpu.VMEM),
            pl.BlockSpec(memory_space=pltpu.SMEM),
        ],
        out_specs=pl.BlockSpec(memory_space=pltpu.VMEM),
        scratch_shapes=[
            pltpu.VMEM((N_DEV, m_per, k), jnp.float8_e4m3fn),
            pltpu.SemaphoreType.DMA((N_DEV - 1,)),
            pltpu.SemaphoreType.DMA((N_DEV - 1,)),
        ],
        compiler_params=pltpu.CompilerParams(collective_id=0),
    )(x8, wb, s)


# device time: 113984 ns/iter; 1.8967x vs baseline; 1.8967x over previous
import jax
import jax.numpy as jnp
from jax import lax
from jax.experimental import pallas as pl
from jax.experimental.pallas import tpu as pltpu

N_DEV = 16


def kernel(x, w_mat, scale_x, scale_w):
    m_per, k = x.shape
    _, n_per = w_mat.shape

    x8 = x.astype(jnp.float8_e4m3fn)
    wb = w_mat.astype(jnp.bfloat16)
    s = (scale_x * scale_w).reshape(1, 1)

    N_CW = N_DEV // 2
    N_CCW = N_DEV - 1 - N_CW

    def body(x_ref, w_ref, s_ref, out_ref, gat_ref,
             cw_send_sems, cw_recv_sems, ccw_send_sems, ccw_recv_sems):
        my = lax.axis_index("i")
        left = lax.rem(my + N_DEV - 1, N_DEV)
        right = lax.rem(my + 1, N_DEV)

        barrier_sem = pltpu.get_barrier_semaphore()
        for nbr in (left, right):
            pl.semaphore_signal(
                barrier_sem, inc=1,
                device_id=(nbr,), device_id_type=pl.DeviceIdType.MESH,
            )
        pl.semaphore_wait(barrier_sem, 2)

        gat_ref[my] = x_ref[...]

        def compute(c):
            xb = gat_ref[c].astype(jnp.bfloat16)
            acc = jnp.dot(xb, w_ref[...], preferred_element_type=jnp.float32)
            y = acc * s_ref[0, 0]
            out_ref[pl.ds(c * m_per, m_per), :] = y * (
                1.0 / (1.0 + jnp.exp(-y))
            )

        def mk(dst_dev, slot, send_sem, recv_sem):
            return pltpu.make_async_remote_copy(
                src_ref=gat_ref.at[slot],
                dst_ref=gat_ref.at[slot],
                send_sem=send_sem,
                recv_sem=recv_sem,
                device_id=(dst_dev,),
                device_id_type=pl.DeviceIdType.MESH,
            )

        cw_d = [None] * N_CW
        ccw_d = [None] * N_CCW

        def cw_issue(h):
            slot = lax.rem(my - h + N_DEV, N_DEV)
            cw_d[h] = mk(right, slot, cw_send_sems.at[h], cw_recv_sems.at[h])
            cw_d[h].start()

        def ccw_issue(h):
            slot = lax.rem(my + h, N_DEV)
            ccw_d[h] = mk(left, slot, ccw_send_sems.at[h], ccw_recv_sems.at[h])
            ccw_d[h].start()

        cw_issue(0)
        ccw_issue(0)
        compute(my)

        for h in range(N_CW):
            cw_d[h].wait_recv()
            if h + 1 < N_CW:
                cw_issue(h + 1)
            if h < N_CCW:
                ccw_d[h].wait_recv()
                if h + 1 < N_CCW:
                    ccw_issue(h + 1)
            compute(lax.rem(my - 1 - h + N_DEV, N_DEV))
            if h < N_CCW:
                compute(lax.rem(my + 1 + h, N_DEV))

        for d in cw_d + ccw_d:
            d.wait_send()

    return pl.pallas_call(
        body,
        out_shape=jax.ShapeDtypeStruct((N_DEV * m_per, n_per), jnp.float32),
        in_specs=[
            pl.BlockSpec(memory_space=pltpu.VMEM),
            pl.BlockSpec(memory_space=pltpu.VMEM),
            pl.BlockSpec(memory_space=pltpu.SMEM),
        ],
        out_specs=pl.BlockSpec(memory_space=pltpu.VMEM),
        scratch_shapes=[
            pltpu.VMEM((N_DEV, m_per, k), jnp.float8_e4m3fn),
            pltpu.SemaphoreType.DMA((N_CW,)),
            pltpu.SemaphoreType.DMA((N_CW,)),
            pltpu.SemaphoreType.DMA((N_CCW,)),
            pltpu.SemaphoreType.DMA((N_CCW,)),
        ],
        compiler_params=pltpu.CompilerParams(collective_id=0),
    )(x8, wb, s)


# device time: 98761 ns/iter; 2.1891x vs baseline; 1.1541x over previous
import jax
import jax.numpy as jnp
from jax import lax
from jax.experimental import pallas as pl
from jax.experimental.pallas import tpu as pltpu

N_DEV = 16


def kernel(x, w_mat, scale_x, scale_w):
    m_per, k = x.shape
    _, n_per = w_mat.shape

    x8 = x.astype(jnp.float8_e4m3fn)
    wb = w_mat.astype(jnp.bfloat16)
    s = (scale_x * scale_w).reshape(1, 1)

    half = m_per // 2
    NH = 2 * N_DEV
    H = NH // 2 - 1

    def body(x_ref, w_ref, s_ref, out_ref, gat_ref,
             cw_send_sems, cw_recv_sems, ccw_send_sems, ccw_recv_sems):
        my = lax.axis_index("i")
        left = lax.rem(my + N_DEV - 1, N_DEV)
        right = lax.rem(my + 1, N_DEV)

        barrier_sem = pltpu.get_barrier_semaphore()
        for nbr in (left, right):
            pl.semaphore_signal(
                barrier_sem, inc=1,
                device_id=(nbr,), device_id_type=pl.DeviceIdType.MESH,
            )
        pl.semaphore_wait(barrier_sem, 2)

        def chunk_slot(c, b):
            return 2 * lax.rem(c + N_DEV, N_DEV) + b

        gat_ref[2 * my] = x_ref[pl.ds(0, half)]
        gat_ref[2 * my + 1] = x_ref[pl.ds(half, half)]

        def compute(s):
            xb = gat_ref[s].astype(jnp.bfloat16)
            acc = jnp.dot(xb, w_ref[...], preferred_element_type=jnp.float32)
            y = acc * s_ref[0, 0]
            out_ref[pl.ds(s * half, half), :] = y * (
                1.0 / (1.0 + jnp.exp(-y))
            )

        S_cw = [chunk_slot(my - j // 2, j % 2) for j in range(H - 1)]
        S_cw += [chunk_slot(my - 7, 0)]
        R_cw = [chunk_slot(my - 1 - j // 2, j % 2) for j in range(H - 1)]
        R_cw += [chunk_slot(my - 8, 0)]
        S_ccw = [chunk_slot(my + j // 2, j % 2) for j in range(H - 1)]
        S_ccw += [chunk_slot(my + 7, 1)]
        R_ccw = [chunk_slot(my + 1 + j // 2, j % 2) for j in range(H - 1)]
        R_ccw += [chunk_slot(my + 8, 1)]

        def mk(dst_dev, slot, send_sem, recv_sem):
            return pltpu.make_async_remote_copy(
                src_ref=gat_ref.at[slot],
                dst_ref=gat_ref.at[slot],
                send_sem=send_sem,
                recv_sem=recv_sem,
                device_id=(dst_dev,),
                device_id_type=pl.DeviceIdType.MESH,
            )

        cw_d = [None] * H
        ccw_d = [None] * H

        def cw_issue(j):
            cw_d[j] = mk(right, S_cw[j], cw_send_sems.at[j],
                         cw_recv_sems.at[j])
            cw_d[j].start()

        def ccw_issue(j):
            ccw_d[j] = mk(left, S_ccw[j], ccw_send_sems.at[j],
                          ccw_recv_sems.at[j])
            ccw_d[j].start()

        cw_issue(0)
        ccw_issue(0)
        cw_issue(1)
        ccw_issue(1)
        compute(2 * my)
        compute(2 * my + 1)

        for j in range(H):
            cw_d[j].wait_recv()
            if j + 2 < H:
                cw_issue(j + 2)
            ccw_d[j].wait_recv()
            if j + 2 < H - 1:
                ccw_issue(j + 2)
            if j == H - 2:
                ccw_issue(H - 1)
            compute(R_cw[j])
            compute(R_ccw[j])

        for d in cw_d + ccw_d:
            d.wait_send()

    return pl.pallas_call(
        body,
        out_shape=jax.ShapeDtypeStruct((N_DEV * m_per, n_per), jnp.float32),
        in_specs=[
            pl.BlockSpec(memory_space=pltpu.VMEM),
            pl.BlockSpec(memory_space=pltpu.VMEM),
            pl.BlockSpec(memory_space=pltpu.SMEM),
        ],
        out_specs=pl.BlockSpec(memory_space=pltpu.VMEM),
        scratch_shapes=[
            pltpu.VMEM((NH, half, k), jnp.float8_e4m3fn),
            pltpu.SemaphoreType.DMA((H,)),
            pltpu.SemaphoreType.DMA((H,)),
            pltpu.SemaphoreType.DMA((H,)),
            pltpu.SemaphoreType.DMA((H,)),
        ],
        compiler_params=pltpu.CompilerParams(collective_id=0),
    )(x8, wb, s)


# device time: 97747 ns/iter; 2.2118x vs baseline; 1.0104x over previous
import jax
import jax.numpy as jnp
from jax import lax
from jax.experimental import pallas as pl
from jax.experimental.pallas import tpu as pltpu

N_DEV = 16


def kernel(x, w_mat, scale_x, scale_w):
    m_per, k = x.shape
    _, n_per = w_mat.shape

    x8 = x.astype(jnp.float8_e4m3fn)
    wb = w_mat.astype(jnp.bfloat16)
    s = (scale_x * scale_w).reshape(1, 1)

    Q = 4
    half = m_per // Q
    NH = Q * N_DEV
    H = NH // 2 - Q // 2

    def body(x_ref, w_ref, s_ref, out_ref, gat_ref,
             cw_send_sems, cw_recv_sems, ccw_send_sems, ccw_recv_sems):
        my = lax.axis_index("i")
        left = lax.rem(my + N_DEV - 1, N_DEV)
        right = lax.rem(my + 1, N_DEV)

        barrier_sem = pltpu.get_barrier_semaphore()
        for nbr in (left, right):
            pl.semaphore_signal(
                barrier_sem, inc=1,
                device_id=(nbr,), device_id_type=pl.DeviceIdType.MESH,
            )
        pl.semaphore_wait(barrier_sem, 2)

        def chunk_slot(c, u):
            return Q * lax.rem(c + N_DEV, N_DEV) + u

        def compute(s, src=None):
            xb = (gat_ref[s] if src is None else src).astype(jnp.bfloat16)
            acc = jnp.dot(xb, w_ref[...], preferred_element_type=jnp.float32)
            y = acc * s_ref[0, 0]
            out_ref[pl.ds(s * half, half), :] = y * (
                1.0 / (1.0 + jnp.exp(-y))
            )

        S_cw = [chunk_slot(my - j // Q, j % Q) for j in range(H - 2)]
        S_cw += [chunk_slot(my - 7, 0), chunk_slot(my - 7, 1)]
        R_cw = [chunk_slot(my - 1 - j // Q, j % Q) for j in range(H - 2)]
        R_cw += [chunk_slot(my - 8, 0), chunk_slot(my - 8, 1)]
        S_ccw = [chunk_slot(my + j // Q, j % Q) for j in range(H - 2)]
        S_ccw += [chunk_slot(my + 7, 2), chunk_slot(my + 7, 3)]
        R_ccw = [chunk_slot(my + 1 + j // Q, j % Q) for j in range(H - 2)]
        R_ccw += [chunk_slot(my + 8, 2), chunk_slot(my + 8, 3)]

        def mk(dst_dev, slot, send_sem, recv_sem, src=None):
            return pltpu.make_async_remote_copy(
                src_ref=gat_ref.at[slot] if src is None else src,
                dst_ref=gat_ref.at[slot],
                send_sem=send_sem,
                recv_sem=recv_sem,
                device_id=(dst_dev,),
                device_id_type=pl.DeviceIdType.MESH,
            )

        cw_d = [None] * H
        ccw_d = [None] * H

        def cw_issue(j, src=None):
            cw_d[j] = mk(right, S_cw[j], cw_send_sems.at[j],
                         cw_recv_sems.at[j], src)
            cw_d[j].start()

        def ccw_issue(j, src=None):
            ccw_d[j] = mk(left, S_ccw[j], ccw_send_sems.at[j],
                          ccw_recv_sems.at[j], src)
            ccw_d[j].start()

        for u in range(Q):
            own = x_ref.at[pl.ds(u * half, half)]
            cw_issue(u, own)
            ccw_issue(u, own)
        for u in range(Q):
            compute(Q * my + u, x_ref[pl.ds(u * half, half)])

        for j in range(H):
            cw_d[j].wait_recv()
            if j + Q < H:
                cw_issue(j + Q)
            ccw_d[j].wait_recv()
            if j + Q < H - 2:
                ccw_issue(j + Q)
            if j in (H - 4, H - 3):
                ccw_issue(j + 2)
            compute(R_cw[j])
            compute(R_ccw[j])

        for d in cw_d + ccw_d:
            d.wait_send()

    return pl.pallas_call(
        body,
        out_shape=jax.ShapeDtypeStruct((N_DEV * m_per, n_per), jnp.float32),
        in_specs=[
            pl.BlockSpec(memory_space=pltpu.VMEM),
            pl.BlockSpec(memory_space=pltpu.VMEM),
            pl.BlockSpec(memory_space=pltpu.SMEM),
        ],
        out_specs=pl.BlockSpec(memory_space=pltpu.VMEM),
        scratch_shapes=[
            pltpu.VMEM((NH, half, k), jnp.float8_e4m3fn),
            pltpu.SemaphoreType.DMA((H,)),
            pltpu.SemaphoreType.DMA((H,)),
            pltpu.SemaphoreType.DMA((H,)),
            pltpu.SemaphoreType.DMA((H,)),
        ],
        compiler_params=pltpu.CompilerParams(collective_id=0),
    )(x8, wb, s)
